# Initial kernel scaffold; baseline (speedup 1.0000x reference)
import jax
import jax.numpy as jnp
from jax.experimental import pallas as pl


def kernel(x, edge_index, batch, W1, b1, W2, b2, W3, b3, Wm1, bm1, Wm2, bm2):
    raise NotImplementedError("write your pallas kernel here")


# speedup vs baseline: 18.7865x; 18.7865x over previous
import functools

import jax
import jax.numpy as jnp
from jax import lax
from jax.experimental import pallas as pl
from jax.experimental.pallas import tpu as pltpu
from jax.experimental.pallas import tpu_sc as plsc

N = 10000
E = 320000
D_IN = 128
H = 64
B = 64

NC = 2
NS = 16
NW = NC * NS
EPW = E // NW
CH = 80
NCHUNK = EPW // CH
RPT = N // NS

_mesh = plsc.VectorSubcoreMesh(
    core_axis_name="c", subcore_axis_name="s", num_cores=NC, num_subcores=NS)
_sc_params = pltpu.CompilerParams(use_tc_tiling_on_sc=False)


@functools.partial(
    pl.kernel,
    out_type=jax.ShapeDtypeStruct((NC, NS, RPT, 16), jnp.float32),
    mesh=_mesh,
    scratch_types=[
        pltpu.VMEM_SHARED((N, 16), jnp.float32),
        pltpu.VMEM((125, 16), jnp.float32),
        pltpu.VMEM((CH, 16), jnp.float32),
        pltpu.VMEM((CH,), jnp.int32),
    ],
    compiler_params=_sc_params,
)
def _sc_degree(dst_hbm, out_hbm, deg_sh, zbuf, ones_v, idx_v):
    c = lax.axis_index("c")
    s = lax.axis_index("s")
    base = (c * NS + s) * EPW
    row0 = s * RPT

    def _zb(i, _):
        zbuf[i, :] = jnp.zeros((16,), jnp.float32)
        return 0
    lax.fori_loop(0, 125, _zb, 0)

    def _ob(i, _):
        ones_v[i, :] = jnp.full((16,), 1.0, jnp.float32)
        return 0
    lax.fori_loop(0, CH, _ob, 0)

    for k in range(RPT // 125):
        pltpu.sync_copy(zbuf, deg_sh.at[pl.ds(row0 + k * 125, 125)])
    plsc.subcore_barrier()

    def _chunk(i, _):
        pltpu.sync_copy(dst_hbm.at[pl.ds(base + i * CH, CH)], idx_v)
        pltpu.sync_copy(ones_v, deg_sh.at[idx_v], add=True)
        return 0
    lax.fori_loop(0, NCHUNK, _chunk, 0)

    plsc.subcore_barrier()
    pltpu.sync_copy(deg_sh.at[pl.ds(row0, RPT)], out_hbm.at[c, s])


@functools.partial(
    pl.kernel,
    out_type=jax.ShapeDtypeStruct((NC, NS, RPT, H), jnp.float32),
    mesh=_mesh,
    scratch_types=[
        pltpu.VMEM_SHARED((N, H), jnp.float32),
        pltpu.VMEM((125, H), jnp.float32),
        pltpu.VMEM((CH,), jnp.int32),
        pltpu.VMEM((CH,), jnp.int32),
        pltpu.VMEM((CH,), jnp.int32),
        pltpu.VMEM((CH,), jnp.int32),
        pltpu.VMEM((CH, H), jnp.float32),
        pltpu.VMEM((CH, H), jnp.float32),
        pltpu.SemaphoreType.DMA,
        pltpu.SemaphoreType.DMA,
    ],
    compiler_params=_sc_params,
)
def _sc_scatter(y_hbm, src_hbm, dst_hbm, out_hbm, acc_sh, zbuf,
                is0, is1, id0, id1, r0, r1, sm0, sm1):
    c = lax.axis_index("c")
    s = lax.axis_index("s")
    base = (c * NS + s) * EPW
    row0 = s * RPT
    IS = (is0, is1)
    ID = (id0, id1)
    R = (r0, r1)
    SM = (sm0, sm1)

    def _zb(i, _):
        for j in range(H // 16):
            zbuf[i, pl.ds(j * 16, 16)] = jnp.zeros((16,), jnp.float32)
        return 0
    lax.fori_loop(0, 125, _zb, 0)
    for k in range(RPT // 125):
        pltpu.sync_copy(zbuf, acc_sh.at[pl.ds(row0 + k * 125, 125)])
    plsc.subcore_barrier()

    def _fire(i, b):
        pltpu.sync_copy(src_hbm.at[pl.ds(base + i * CH, CH)], IS[b])
        pltpu.sync_copy(dst_hbm.at[pl.ds(base + i * CH, CH)], ID[b])
        pltpu.async_copy(y_hbm.at[IS[b]], R[b], SM[b])

    _fire(0, 0)
    _fire(1, 1)

    def _body(g, _):
        for b in range(2):
            i = 2 * g + b

            @pl.when(i < NCHUNK)
            def _():
                pltpu.make_async_copy(y_hbm.at[IS[b]], R[b], SM[b]).wait()
                pltpu.sync_copy(R[b], acc_sh.at[ID[b]], add=True)

                @pl.when(i + 2 < NCHUNK)
                def _():
                    _fire(i + 2, b)
        return 0
    lax.fori_loop(0, (NCHUNK + 1) // 2, _body, 0)

    plsc.subcore_barrier()
    pltpu.sync_copy(acc_sh.at[pl.ds(row0, RPT)], out_hbm.at[c, s])


RB = 1000
NRB = N // RB


def _prep_body(x_ref, w_ref, d_ref, y_ref, dv_ref):
    d = d_ref[...]
    deg = (d[0] + d[1])[:, 0:1] + 1.0
    dinv = lax.rsqrt(jnp.maximum(deg, 1.0))
    dv_ref[...] = dinv
    xw = jnp.dot(x_ref[...], w_ref[...], preferred_element_type=jnp.float32)
    y_ref[...] = xw * dinv


_tc_prep = pl.pallas_call(
    _prep_body,
    grid=(NRB,),
    in_specs=[
        pl.BlockSpec((RB, D_IN), lambda i: (i, 0)),
        pl.BlockSpec((D_IN, H), lambda i: (0, 0)),
        pl.BlockSpec((NC, RB, 16), lambda i: (0, i, 0)),
    ],
    out_specs=[
        pl.BlockSpec((RB, H), lambda i: (i, 0)),
        pl.BlockSpec((RB, 1), lambda i: (i, 0)),
    ],
    out_shape=[
        jax.ShapeDtypeStruct((N, H), jnp.float32),
        jax.ShapeDtypeStruct((N, 1), jnp.float32),
    ],
)


def _combine_body(a_ref, y_ref, dv_ref, b_ref, w_ref, o_ref):
    a = a_ref[...]
    sm = a[0] + a[1] + y_ref[...]
    dv = dv_ref[...]
    h = jnp.maximum(sm * dv + b_ref[...], 0.0)
    o_ref[...] = jnp.dot(h, w_ref[...], preferred_element_type=jnp.float32) * dv


_tc_combine = pl.pallas_call(
    _combine_body,
    grid=(NRB,),
    in_specs=[
        pl.BlockSpec((NC, RB, H), lambda i: (0, i, 0)),
        pl.BlockSpec((RB, H), lambda i: (i, 0)),
        pl.BlockSpec((RB, 1), lambda i: (i, 0)),
        pl.BlockSpec((1, H), lambda i: (0, 0)),
        pl.BlockSpec((H, H), lambda i: (0, 0)),
    ],
    out_specs=pl.BlockSpec((RB, H), lambda i: (i, 0)),
    out_shape=jax.ShapeDtypeStruct((N, H), jnp.float32),
)


def _final_body(a_ref, y_ref, dv_ref, b_ref, bat_ref, wm1_ref, bm1_ref,
                wm2_ref, bm2_ref, o_ref, pool_ref):
    i = pl.program_id(0)

    @pl.when(i == 0)
    def _():
        pool_ref[...] = jnp.zeros((B, H + 1), jnp.float32)

    a = a_ref[...]
    sm = a[0] + a[1] + y_ref[...]
    h = jnp.maximum(sm * dv_ref[...] + b_ref[...], 0.0)
    he = jnp.concatenate([h, jnp.ones((RB, 1), jnp.float32)], axis=1)
    bid = lax.broadcasted_iota(jnp.int32, (RB, B), 1)
    oh = (bat_ref[...] == bid).astype(jnp.float32)
    pool_ref[...] += lax.dot_general(
        oh, he, (((0,), (0,)), ((), ())), preferred_element_type=jnp.float32)

    @pl.when(i == NRB - 1)
    def _():
        p = pool_ref[...]
        cnt = jnp.maximum(p[:, H:H + 1], 1.0)
        mean = p[:, 0:H] / cnt
        hid = jnp.maximum(
            jnp.dot(mean, wm1_ref[...], preferred_element_type=jnp.float32)
            + bm1_ref[...], 0.0)
        o_ref[...] = jnp.dot(
            hid, wm2_ref[...], preferred_element_type=jnp.float32) + bm2_ref[...]


_tc_final = pl.pallas_call(
    _final_body,
    grid=(NRB,),
    in_specs=[
        pl.BlockSpec((NC, RB, H), lambda i: (0, i, 0)),
        pl.BlockSpec((RB, H), lambda i: (i, 0)),
        pl.BlockSpec((RB, 1), lambda i: (i, 0)),
        pl.BlockSpec((1, H), lambda i: (0, 0)),
        pl.BlockSpec((RB, 1), lambda i: (i, 0)),
        pl.BlockSpec((H, H // 2), lambda i: (0, 0)),
        pl.BlockSpec((1, H // 2), lambda i: (0, 0)),
        pl.BlockSpec((H // 2, 1), lambda i: (0, 0)),
        pl.BlockSpec((1, 1), lambda i: (0, 0)),
    ],
    out_specs=pl.BlockSpec((B, 1), lambda i: (0, 0)),
    out_shape=jax.ShapeDtypeStruct((B, 1), jnp.float32),
    scratch_shapes=[pltpu.VMEM((B, H + 1), jnp.float32)],
)


def kernel(x, edge_index, batch, W1, b1, W2, b2, W3, b3, Wm1, bm1, Wm2, bm2):
    src = edge_index[0]
    dst = edge_index[1]

    degp = _sc_degree(dst).reshape(NC, N, 16)
    y1, dinv = _tc_prep(x, W1, degp)

    acc1 = _sc_scatter(y1, src, dst).reshape(NC, N, H)
    y2 = _tc_combine(acc1, y1, dinv, b1.reshape(1, H), W2)

    acc2 = _sc_scatter(y2, src, dst).reshape(NC, N, H)
    y3 = _tc_combine(acc2, y2, dinv, b2.reshape(1, H), W3)

    acc3 = _sc_scatter(y3, src, dst).reshape(NC, N, H)
    out = _tc_final(acc3, y3, dinv, b3.reshape(1, H),
                    batch.reshape(N, 1),
                    Wm1, bm1.reshape(1, H // 2), Wm2, bm2.reshape(1, 1))
    return out
